# baseline (speedup 1.0000x reference)
import jax
import jax.numpy as jnp
from jax import lax
from jax.experimental import pallas as pl
from jax.experimental.pallas import tpu as pltpu
from jax.experimental.pallas import tpu_sc as plsc

B, C, H, W = 2, 96, 512, 512
HW = H * W
NS = 16
SLICE = HW // NS
CH = 512
NCH = SLICE // CH
NV = CH // 16
L = 16
PAD = 520
PLANE = HW + 1048
CORNER_OFF = (0, 1, W, W + 1)


def _floor16(v):
    t = v.astype(jnp.int32)
    tf = t.astype(jnp.float32)
    adj = tf > v
    return jnp.where(adj, t - 1, t), jnp.where(adj, tf - 1.0, tf)


def _sc_body(ten_hbm, flow_hbm, mask_hbm, out_hbm,
             acc_sh, den_sh,
             base_c, w00_c, w10_c, w01_c, w11_c,
             src_b,
             ia0, ia1, ia2, ia3, ib0, ib1, ib2, ib3,
             va0, va1, va2, va3, vb0, vb1, vb2, vb3,
             zz_b,
             sem_l0, sem_l1, sem_a, sem_b, sem_w0, sem_w1):
    b = lax.axis_index("c")
    s = lax.axis_index("s")
    s0 = s * SLICE
    wrefs = (w00_c, w10_c, w01_c, w11_c)
    idx_sets = ((ia0, ia1, ia2, ia3), (ib0, ib1, ib2, ib3))
    val_sets = ((va0, va1, va2, va3), (vb0, vb1, vb2, vb3))
    sc_sems = (sem_a, sem_b)
    ld_sems = (sem_l0, sem_l1)
    wr_sems = (sem_w0, sem_w1)

    def meta_chunk(j, _):
        off = j * CH
        pltpu.sync_copy(mask_hbm.at[b, pl.ds(s0 + off, CH)], va0)

        def mvec(i, _):
            w11_c[pl.ds(off + i * L, L)] = jnp.exp(va0[pl.ds(i * L, L)])
            return 0

        lax.fori_loop(0, NV, mvec, 0, unroll=4)

        pltpu.sync_copy(flow_hbm.at[2 * b, pl.ds(s0 + off, CH)], va0)
        pltpu.sync_copy(flow_hbm.at[2 * b + 1, pl.ds(s0 + off, CH)], va1)

        def vec(i, _):
            sl = pl.ds(i * L, L)
            gsl = pl.ds(off + i * L, L)
            m = w11_c[gsl]
            p = (s0 + off + i * L) + lax.iota(jnp.int32, L)
            xg = jnp.bitwise_and(p, W - 1).astype(jnp.float32)
            yg = jnp.right_shift(p, 9).astype(jnp.float32)
            fx = jnp.minimum(jnp.maximum(xg + va0[sl], -2.0), W + 1.0)
            fy = jnp.minimum(jnp.maximum(yg + va1[sl], -2.0), H + 1.0)
            x0, x0f = _floor16(fx)
            y0, y0f = _floor16(fy)
            frx = fx - x0f
            fry = fy - y0f
            zero = jnp.zeros((L,), jnp.float32)
            vx0 = (x0 >= 0) & (x0 < W)
            vx1 = (x0 >= -1) & (x0 < W - 1)
            vy0 = (y0 >= 0) & (y0 < H)
            vy1 = (y0 >= -1) & (y0 < H - 1)
            wx0 = 1.0 - frx
            wy0 = 1.0 - fry
            bi = y0 * W + x0 + PAD
            base_c[gsl] = jnp.minimum(jnp.maximum(bi, 0), HW + PAD + 7)
            w00_c[gsl] = jnp.where(vx0 & vy0, m * (wx0 * wy0), zero)
            w10_c[gsl] = jnp.where(vx1 & vy0, m * (frx * wy0), zero)
            w01_c[gsl] = jnp.where(vx0 & vy1, m * (wx0 * fry), zero)
            w11_c[gsl] = jnp.where(vx1 & vy1, m * (frx * fry), zero)
            return 0

        lax.fori_loop(0, NV, vec, 0, unroll=2)
        return 0

    lax.fori_loop(0, NCH, meta_chunk, 0)

    def zvec(i, _):
        zz_b[pl.ds(i * L, L)] = jnp.zeros((L,), jnp.float32)
        return 0

    lax.fori_loop(0, NV, zvec, 0)

    def zero_plane(plane):
        def zc(j, _):
            pltpu.sync_copy(zz_b, plane.at[pl.ds(PAD + s0 + j * CH, CH)])
            return 0
        lax.fori_loop(0, NCH, zc, 0)

    def scatter_wait_set(st, plane):
        for k in range(4):
            pltpu.make_async_copy(val_sets[st][k],
                                  plane.at[idx_sets[st][k]],
                                  sc_sems[st]).wait()

    def do_chunk(j, plane, with_src, st, soff, first):
        off = j * CH
        iref, vref = idx_sets[st], val_sets[st]
        if not first:
            scatter_wait_set(st, plane)

        @plsc.parallel_loop(0, NV, 1, unroll=4)
        def _(i):
            sl = pl.ds(i * L, L)
            gsl = pl.ds(off + i * L, L)
            bb = base_c[gsl]
            if with_src:
                sv = src_b[pl.ds(soff + i * L, L)]
            for k in range(4):
                iref[k][sl] = bb + CORNER_OFF[k]
                wv = wrefs[k][gsl]
                vref[k][sl] = wv * sv if with_src else wv

        for k in range(4):
            pltpu.async_copy(vref[k], plane.at[iref[k]], sc_sems[st],
                             add=True)

    def fire_load(j, row, sslot):
        pltpu.async_copy(ten_hbm.at[row, pl.ds(s0 + j * CH, CH)],
                         src_b.at[pl.ds(sslot * CH, CH)], ld_sems[sslot])

    def wait_load(j, row, sslot):
        pltpu.make_async_copy(ten_hbm.at[row, pl.ds(s0 + j * CH, CH)],
                              src_b.at[pl.ds(sslot * CH, CH)],
                              ld_sems[sslot]).wait()

    def scatter_plane(plane, row, with_src):
        if with_src:
            fire_load(0, row, 0)

        def pair(t, first):
            j0 = 2 * t
            j1 = 2 * t + 1
            if with_src:
                wait_load(j0, row, 0)
                fire_load(j1, row, 1)
            do_chunk(j0, plane, with_src, 0, 0, first)
            if with_src:
                wait_load(j1, row, 1)
                fire_load(lax.rem(j0 + 2, NCH), row, 0)
            do_chunk(j1, plane, with_src, 1, CH, first)
            return 0

        pair(0, True)
        lax.fori_loop(1, NCH // 2, lambda t, _: pair(t, False), 0)
        if with_src:
            wait_load(0, row, 0)
        scatter_wait_set(0, plane)
        scatter_wait_set(1, plane)

    zero_plane(den_sh)
    zero_plane(acc_sh)
    pass
    scatter_plane(den_sh, 0, with_src=False)
    pass

    ost = ((va0, va1), (va2, va3))

    def out_chunk(j, row, u, first):
        onum, oden = ost[u]
        dsl = pl.ds(PAD + s0 + j * CH, CH)
        osl = pl.ds(s0 + j * CH, CH)
        if not first:
            pltpu.make_async_copy(onum, out_hbm.at[row, osl],
                                  wr_sems[u]).wait()
        pltpu.sync_copy(acc_sh.at[dsl], onum)
        pltpu.sync_copy(den_sh.at[dsl], oden)

        @plsc.parallel_loop(0, NV, 1, unroll=8)
        def _(i):
            sl = pl.ds(i * L, L)
            onum[sl] = onum[sl] / (oden[sl] + 1e-7)

        pltpu.async_copy(onum, out_hbm.at[row, osl], wr_sems[u])
        pltpu.sync_copy(zz_b, acc_sh.at[dsl])

    def channel(c, _):
        row = b * C + c
        scatter_plane(acc_sh, row, with_src=True)
        pass

        def opair(t, first):
            out_chunk(2 * t, row, 0, first)
            out_chunk(2 * t + 1, row, 1, first)
            return 0

        opair(0, True)
        lax.fori_loop(1, NCH // 2, lambda t, _: opair(t, False), 0)
        for u, j in ((0, NCH - 2), (1, NCH - 1)):
            pltpu.make_async_copy(ost[u][0],
                                  out_hbm.at[row, pl.ds(s0 + j * CH, CH)],
                                  wr_sems[u]).wait()
        pass
        return 0

    lax.fori_loop(0, C, channel, 0)


@jax.jit
def _softsplat_sc(ten2d, flow2d, mask2d):
    mesh = plsc.VectorSubcoreMesh(core_axis_name="c", subcore_axis_name="s")
    fn = pl.kernel(
        _sc_body,
        mesh=mesh,
        out_type=jax.ShapeDtypeStruct((B * C, HW), jnp.float32),
        scratch_types=[
            pltpu.VMEM_SHARED((PLANE,), jnp.float32),
            pltpu.VMEM_SHARED((PLANE,), jnp.float32),
            pltpu.VMEM((SLICE,), jnp.int32),
            pltpu.VMEM((SLICE,), jnp.float32),
            pltpu.VMEM((SLICE,), jnp.float32),
            pltpu.VMEM((SLICE,), jnp.float32),
            pltpu.VMEM((SLICE,), jnp.float32),
            pltpu.VMEM((2 * CH,), jnp.float32),
            *[pltpu.VMEM((CH,), jnp.int32) for _ in range(8)],
            *[pltpu.VMEM((CH,), jnp.float32) for _ in range(8)],
            pltpu.VMEM((CH,), jnp.float32),
            pltpu.SemaphoreType.DMA,
            pltpu.SemaphoreType.DMA,
            pltpu.SemaphoreType.DMA,
            pltpu.SemaphoreType.DMA,
            pltpu.SemaphoreType.DMA,
            pltpu.SemaphoreType.DMA,
        ],
    )
    return fn(ten2d, flow2d, mask2d)


def kernel(tenInput, tenFlow, importance_mask):
    ten2d = tenInput.reshape(B * C, HW)
    flow2d = tenFlow.reshape(B * 2, HW)
    mask2d = importance_mask.reshape(B, HW)
    out = _softsplat_sc(ten2d, flow2d, mask2d)
    return out.reshape(B, C, H, W)


# speedup vs baseline: 1.0263x; 1.0263x over previous
import jax
import jax.numpy as jnp
from jax import lax
from jax.experimental import pallas as pl
from jax.experimental.pallas import tpu as pltpu
from jax.experimental.pallas import tpu_sc as plsc

B, C, H, W = 2, 96, 512, 512
HW = H * W
NS = 16
SLICE = HW // NS
CH = 512
NCH = SLICE // CH
NV = CH // 16
L = 16
PAD = 520
PLANE = HW + 1048
CORNER_OFF = (0, 1, W, W + 1)


def _floor16(v):
    t = v.astype(jnp.int32)
    tf = t.astype(jnp.float32)
    adj = tf > v
    return jnp.where(adj, t - 1, t), jnp.where(adj, tf - 1.0, tf)


def _sc_body(ten_hbm, flow_hbm, mask_hbm, out_hbm,
             acc_sh, den_sh,
             base_c, w00_c, w10_c, w01_c, w11_c,
             src_b,
             ia0, ia1, ia2, ia3, ib0, ib1, ib2, ib3,
             va0, va1, va2, va3, vb0, vb1, vb2, vb3,
             zz_b,
             sem_l0, sem_l1, sem_a, sem_b, sem_w0, sem_w1):
    b = lax.axis_index("c")
    s = lax.axis_index("s")
    s0 = s * SLICE
    wrefs = (w00_c, w10_c, w01_c, w11_c)
    idx_sets = ((ia0, ia1, ia2, ia3), (ib0, ib1, ib2, ib3))
    val_sets = ((va0, va1, va2, va3), (vb0, vb1, vb2, vb3))
    sc_sems = (sem_a, sem_b)
    ld_sems = (sem_l0, sem_l1)
    wr_sems = (sem_w0, sem_w1)

    def meta_chunk(j, _):
        off = j * CH
        pltpu.sync_copy(mask_hbm.at[b, pl.ds(s0 + off, CH)], va0)

        def mvec(i, _):
            w11_c[pl.ds(off + i * L, L)] = jnp.exp(va0[pl.ds(i * L, L)])
            return 0

        lax.fori_loop(0, NV, mvec, 0, unroll=4)

        pltpu.sync_copy(flow_hbm.at[2 * b, pl.ds(s0 + off, CH)], va0)
        pltpu.sync_copy(flow_hbm.at[2 * b + 1, pl.ds(s0 + off, CH)], va1)

        def vec(i, _):
            sl = pl.ds(i * L, L)
            gsl = pl.ds(off + i * L, L)
            m = w11_c[gsl]
            p = (s0 + off + i * L) + lax.iota(jnp.int32, L)
            xg = jnp.bitwise_and(p, W - 1).astype(jnp.float32)
            yg = jnp.right_shift(p, 9).astype(jnp.float32)
            fx = jnp.minimum(jnp.maximum(xg + va0[sl], -2.0), W + 1.0)
            fy = jnp.minimum(jnp.maximum(yg + va1[sl], -2.0), H + 1.0)
            x0, x0f = _floor16(fx)
            y0, y0f = _floor16(fy)
            frx = fx - x0f
            fry = fy - y0f
            zero = jnp.zeros((L,), jnp.float32)
            vx0 = (x0 >= 0) & (x0 < W)
            vx1 = (x0 >= -1) & (x0 < W - 1)
            vy0 = (y0 >= 0) & (y0 < H)
            vy1 = (y0 >= -1) & (y0 < H - 1)
            wx0 = 1.0 - frx
            wy0 = 1.0 - fry
            bi = y0 * W + x0 + PAD
            base_c[gsl] = jnp.minimum(jnp.maximum(bi, 0), HW + PAD + 7)
            w00_c[gsl] = jnp.where(vx0 & vy0, m * (wx0 * wy0), zero)
            w10_c[gsl] = jnp.where(vx1 & vy0, m * (frx * wy0), zero)
            w01_c[gsl] = jnp.where(vx0 & vy1, m * (wx0 * fry), zero)
            w11_c[gsl] = jnp.where(vx1 & vy1, m * (frx * fry), zero)
            return 0

        lax.fori_loop(0, NV, vec, 0, unroll=2)
        return 0

    lax.fori_loop(0, NCH, meta_chunk, 0)

    def zvec(i, _):
        zz_b[pl.ds(i * L, L)] = jnp.zeros((L,), jnp.float32)
        return 0

    lax.fori_loop(0, NV, zvec, 0)

    def zero_plane(plane):
        def zc(j, _):
            pltpu.sync_copy(zz_b, plane.at[pl.ds(PAD + s0 + j * CH, CH)])
            return 0
        lax.fori_loop(0, NCH, zc, 0)

    def scatter_wait_set(st, plane):
        for k in range(4):
            pltpu.make_async_copy(val_sets[st][k],
                                  plane.at[idx_sets[st][k]],
                                  sc_sems[st]).wait()

    def do_chunk(j, plane, with_src, st, soff, first):
        off = j * CH
        iref, vref = idx_sets[st], val_sets[st]
        if not first:
            scatter_wait_set(st, plane)

        @plsc.parallel_loop(0, NV, 1, unroll=8)
        def _(i):
            sl = pl.ds(pl.multiple_of(i * L, L), L)
            gsl = pl.ds(pl.multiple_of(off + i * L, L), L)
            bb = base_c[gsl]
            if with_src:
                sv = src_b[pl.ds(pl.multiple_of(soff + i * L, L), L)]
            for k in range(4):
                iref[k][sl] = bb + CORNER_OFF[k]
                wv = wrefs[k][gsl]
                vref[k][sl] = wv * sv if with_src else wv

        for k in range(4):
            pltpu.async_copy(vref[k], plane.at[iref[k]], sc_sems[st],
                             add=True)

    def fire_load(j, row, sslot):
        pltpu.async_copy(ten_hbm.at[row, pl.ds(s0 + j * CH, CH)],
                         src_b.at[pl.ds(sslot * CH, CH)], ld_sems[sslot])

    def wait_load(j, row, sslot):
        pltpu.make_async_copy(ten_hbm.at[row, pl.ds(s0 + j * CH, CH)],
                              src_b.at[pl.ds(sslot * CH, CH)],
                              ld_sems[sslot]).wait()

    def scatter_plane(plane, row, with_src):
        if with_src:
            fire_load(0, row, 0)

        def pair(t, first):
            j0 = 2 * t
            j1 = 2 * t + 1
            if with_src:
                wait_load(j0, row, 0)
                fire_load(j1, row, 1)
            do_chunk(j0, plane, with_src, 0, 0, first)
            if with_src:
                wait_load(j1, row, 1)
                fire_load(lax.rem(j0 + 2, NCH), row, 0)
            do_chunk(j1, plane, with_src, 1, CH, first)
            return 0

        pair(0, True)
        lax.fori_loop(1, NCH // 2, lambda t, _: pair(t, False), 0)
        if with_src:
            wait_load(0, row, 0)
        scatter_wait_set(0, plane)
        scatter_wait_set(1, plane)

    zero_plane(den_sh)
    zero_plane(acc_sh)
    plsc.subcore_barrier()
    scatter_plane(den_sh, 0, with_src=False)
    plsc.subcore_barrier()

    ost = ((va0, va1), (va2, va3))

    def out_chunk(j, row, u, first):
        onum, oden = ost[u]
        dsl = pl.ds(PAD + s0 + j * CH, CH)
        osl = pl.ds(s0 + j * CH, CH)
        if not first:
            pltpu.make_async_copy(onum, out_hbm.at[row, osl],
                                  wr_sems[u]).wait()
        pltpu.sync_copy(acc_sh.at[dsl], onum)
        pltpu.sync_copy(den_sh.at[dsl], oden)

        @plsc.parallel_loop(0, NV, 1, unroll=8)
        def _(i):
            sl = pl.ds(pl.multiple_of(i * L, L), L)
            onum[sl] = onum[sl] / (oden[sl] + 1e-7)

        pltpu.async_copy(onum, out_hbm.at[row, osl], wr_sems[u])
        pltpu.sync_copy(zz_b, acc_sh.at[dsl])

    def channel(c, _):
        row = b * C + c
        scatter_plane(acc_sh, row, with_src=True)
        plsc.subcore_barrier()

        def opair(t, first):
            out_chunk(2 * t, row, 0, first)
            out_chunk(2 * t + 1, row, 1, first)
            return 0

        opair(0, True)
        lax.fori_loop(1, NCH // 2, lambda t, _: opair(t, False), 0)
        for u, j in ((0, NCH - 2), (1, NCH - 1)):
            pltpu.make_async_copy(ost[u][0],
                                  out_hbm.at[row, pl.ds(s0 + j * CH, CH)],
                                  wr_sems[u]).wait()
        plsc.subcore_barrier()
        return 0

    lax.fori_loop(0, C, channel, 0)


@jax.jit
def _softsplat_sc(ten2d, flow2d, mask2d):
    mesh = plsc.VectorSubcoreMesh(core_axis_name="c", subcore_axis_name="s")
    fn = pl.kernel(
        _sc_body,
        mesh=mesh,
        out_type=jax.ShapeDtypeStruct((B * C, HW), jnp.float32),
        scratch_types=[
            pltpu.VMEM_SHARED((PLANE,), jnp.float32),
            pltpu.VMEM_SHARED((PLANE,), jnp.float32),
            pltpu.VMEM((SLICE,), jnp.int32),
            pltpu.VMEM((SLICE,), jnp.float32),
            pltpu.VMEM((SLICE,), jnp.float32),
            pltpu.VMEM((SLICE,), jnp.float32),
            pltpu.VMEM((SLICE,), jnp.float32),
            pltpu.VMEM((2 * CH,), jnp.float32),
            *[pltpu.VMEM((CH,), jnp.int32) for _ in range(8)],
            *[pltpu.VMEM((CH,), jnp.float32) for _ in range(8)],
            pltpu.VMEM((CH,), jnp.float32),
            pltpu.SemaphoreType.DMA,
            pltpu.SemaphoreType.DMA,
            pltpu.SemaphoreType.DMA,
            pltpu.SemaphoreType.DMA,
            pltpu.SemaphoreType.DMA,
            pltpu.SemaphoreType.DMA,
        ],
    )
    return fn(ten2d, flow2d, mask2d)


def kernel(tenInput, tenFlow, importance_mask):
    ten2d = tenInput.reshape(B * C, HW)
    flow2d = tenFlow.reshape(B * 2, HW)
    mask2d = importance_mask.reshape(B, HW)
    out = _softsplat_sc(ten2d, flow2d, mask2d)
    return out.reshape(B, C, H, W)


# speedup vs baseline: 1.3163x; 1.2825x over previous
import jax
import jax.numpy as jnp
from jax import lax
from jax.experimental import pallas as pl
from jax.experimental.pallas import tpu as pltpu
from jax.experimental.pallas import tpu_sc as plsc

B, C, H, W = 2, 96, 512, 512
HW = H * W
NS = 16
SLICE = HW // NS
CH = 512
NCH = SLICE // CH
NV = CH // 16
L = 16
PAD = 520
PLANE = HW + 1048
CORNER_OFF = (0, 1, W, W + 1)


def _floor16(v):
    t = v.astype(jnp.int32)
    tf = t.astype(jnp.float32)
    adj = tf > v
    return jnp.where(adj, t - 1, t), jnp.where(adj, tf - 1.0, tf)


def _sc_body(ten_hbm, flow_hbm, mask_hbm, out_hbm,
             acc_sh, den_sh,
             base_c, w00_c, w10_c, w01_c, w11_c,
             src_b,
             ia0, ia1, ia2, ia3, ib0, ib1, ib2, ib3,
             va0, va1, va2, va3, vb0, vb1, vb2, vb3,
             zz_b,
             sem_l0, sem_l1, sem_a, sem_b, sem_w0, sem_w1,
             sem_r0, sem_r1, sem_z):
    b = lax.axis_index("c")
    s = lax.axis_index("s")
    s0 = s * SLICE
    wrefs = (w00_c, w10_c, w01_c, w11_c)
    idx_sets = ((ia0, ia1, ia2, ia3), (ib0, ib1, ib2, ib3))
    val_sets = ((va0, va1, va2, va3), (vb0, vb1, vb2, vb3))
    sc_sems = (sem_a, sem_b)
    ld_sems = (sem_l0, sem_l1)
    wr_sems = (sem_w0, sem_w1)

    def meta_chunk(j, _):
        off = j * CH
        pltpu.sync_copy(mask_hbm.at[b, pl.ds(s0 + off, CH)], va0)

        def mvec(i, _):
            w11_c[pl.ds(off + i * L, L)] = jnp.exp(va0[pl.ds(i * L, L)])
            return 0

        lax.fori_loop(0, NV, mvec, 0, unroll=4)

        pltpu.sync_copy(flow_hbm.at[2 * b, pl.ds(s0 + off, CH)], va0)
        pltpu.sync_copy(flow_hbm.at[2 * b + 1, pl.ds(s0 + off, CH)], va1)

        def vec(i, _):
            sl = pl.ds(i * L, L)
            gsl = pl.ds(off + i * L, L)
            m = w11_c[gsl]
            p = (s0 + off + i * L) + lax.iota(jnp.int32, L)
            xg = jnp.bitwise_and(p, W - 1).astype(jnp.float32)
            yg = jnp.right_shift(p, 9).astype(jnp.float32)
            fx = jnp.minimum(jnp.maximum(xg + va0[sl], -2.0), W + 1.0)
            fy = jnp.minimum(jnp.maximum(yg + va1[sl], -2.0), H + 1.0)
            x0, x0f = _floor16(fx)
            y0, y0f = _floor16(fy)
            frx = fx - x0f
            fry = fy - y0f
            zero = jnp.zeros((L,), jnp.float32)
            vx0 = (x0 >= 0) & (x0 < W)
            vx1 = (x0 >= -1) & (x0 < W - 1)
            vy0 = (y0 >= 0) & (y0 < H)
            vy1 = (y0 >= -1) & (y0 < H - 1)
            wx0 = 1.0 - frx
            wy0 = 1.0 - fry
            bi = y0 * W + x0 + PAD
            base_c[gsl] = jnp.minimum(jnp.maximum(bi, 0), HW + PAD + 7)
            w00_c[gsl] = jnp.where(vx0 & vy0, m * (wx0 * wy0), zero)
            w10_c[gsl] = jnp.where(vx1 & vy0, m * (frx * wy0), zero)
            w01_c[gsl] = jnp.where(vx0 & vy1, m * (wx0 * fry), zero)
            w11_c[gsl] = jnp.where(vx1 & vy1, m * (frx * fry), zero)
            return 0

        lax.fori_loop(0, NV, vec, 0, unroll=2)
        return 0

    lax.fori_loop(0, NCH, meta_chunk, 0)

    def zvec(i, _):
        zz_b[pl.ds(i * L, L)] = jnp.zeros((L,), jnp.float32)
        return 0

    lax.fori_loop(0, NV, zvec, 0)

    def zero_plane(plane):
        def zc(j, _):
            pltpu.sync_copy(zz_b, plane.at[pl.ds(PAD + s0 + j * CH, CH)])
            return 0
        lax.fori_loop(0, NCH, zc, 0)

    def scatter_wait_set(st, plane):
        for k in range(4):
            pltpu.make_async_copy(val_sets[st][k],
                                  plane.at[idx_sets[st][k]],
                                  sc_sems[st]).wait()

    def do_chunk(j, plane, with_src, st, soff, first):
        off = j * CH
        iref, vref = idx_sets[st], val_sets[st]
        if not first:
            scatter_wait_set(st, plane)

        @plsc.parallel_loop(0, NV, 1, unroll=8)
        def _(i):
            sl = pl.ds(pl.multiple_of(i * L, L), L)
            gsl = pl.ds(pl.multiple_of(off + i * L, L), L)
            bb = base_c[gsl]
            if with_src:
                sv = src_b[pl.ds(pl.multiple_of(soff + i * L, L), L)]
            for k in range(4):
                iref[k][sl] = bb + CORNER_OFF[k]
                wv = wrefs[k][gsl]
                vref[k][sl] = wv * sv if with_src else wv

        for k in range(4):
            pltpu.async_copy(vref[k], plane.at[iref[k]], sc_sems[st],
                             add=True)

    def fire_load(j, row, sslot):
        pltpu.async_copy(ten_hbm.at[row, pl.ds(s0 + j * CH, CH)],
                         src_b.at[pl.ds(sslot * CH, CH)], ld_sems[sslot])

    def wait_load(j, row, sslot):
        pltpu.make_async_copy(ten_hbm.at[row, pl.ds(s0 + j * CH, CH)],
                              src_b.at[pl.ds(sslot * CH, CH)],
                              ld_sems[sslot]).wait()

    def scatter_plane(plane, row, with_src):
        if with_src:
            fire_load(0, row, 0)

        def pair(t, first):
            j0 = 2 * t
            j1 = 2 * t + 1
            if with_src:
                wait_load(j0, row, 0)
                fire_load(j1, row, 1)
            do_chunk(j0, plane, with_src, 0, 0, first)
            if with_src:
                wait_load(j1, row, 1)
                fire_load(lax.rem(j0 + 2, NCH), row, 0)
            do_chunk(j1, plane, with_src, 1, CH, first)
            return 0

        pair(0, True)
        lax.fori_loop(1, NCH // 2, lambda t, _: pair(t, False), 0)
        if with_src:
            wait_load(0, row, 0)
        scatter_wait_set(0, plane)
        scatter_wait_set(1, plane)

    zero_plane(den_sh)
    zero_plane(acc_sh)
    plsc.subcore_barrier()
    scatter_plane(den_sh, 0, with_src=False)
    plsc.subcore_barrier()

    rd_stage = ((va0, va1), (va2, va3))
    res_stage = (vb0, vb1)
    rd_sems = (sem_r0, sem_r1)

    def fire_reads(j, st):
        dsl = pl.ds(PAD + s0 + j * CH, CH)
        pltpu.async_copy(acc_sh.at[dsl], rd_stage[st][0], rd_sems[st])
        pltpu.async_copy(den_sh.at[dsl], rd_stage[st][1], rd_sems[st])

    def drain_reads(st):
        pltpu.make_async_copy(ten_hbm.at[0, pl.ds(0, 2 * CH)], src_b,
                              rd_sems[st]).wait()

    def out_chunk(j, st, row, fire_next, first_pair):
        osl = pl.ds(s0 + j * CH, CH)
        drain_reads(st)
        if not first_pair:
            pltpu.make_async_copy(res_stage[st], out_hbm.at[row, osl],
                                  wr_sems[st]).wait()
        anum, aden = rd_stage[st]
        res = res_stage[st]

        @plsc.parallel_loop(0, NV, 1, unroll=8)
        def _(i):
            sl = pl.ds(pl.multiple_of(i * L, L), L)
            res[sl] = anum[sl] / (aden[sl] + 1e-7)

        pltpu.async_copy(res, out_hbm.at[row, osl], wr_sems[st])
        pltpu.async_copy(zz_b, acc_sh.at[pl.ds(PAD + s0 + j * CH, CH)],
                         sem_z)
        if fire_next:
            fire_reads(j + 2, st)

    def channel(c, _):
        row = b * C + c
        scatter_plane(acc_sh, row, with_src=True)
        plsc.subcore_barrier()

        fire_reads(0, 0)
        fire_reads(1, 1)

        def opair(t, first_pair, fire_next):
            out_chunk(2 * t, 0, row, fire_next, first_pair)
            out_chunk(2 * t + 1, 1, row, fire_next, first_pair)
            return 0

        opair(0, True, True)
        lax.fori_loop(1, NCH // 2 - 1, lambda t, _: opair(t, False, True), 0)
        opair(NCH // 2 - 1, False, False)
        for st, j in ((0, NCH - 2), (1, NCH - 1)):
            pltpu.make_async_copy(res_stage[st],
                                  out_hbm.at[row, pl.ds(s0 + j * CH, CH)],
                                  wr_sems[st]).wait()
        pltpu.make_async_copy(ten_hbm.at[0, pl.ds(0, SLICE)], base_c,
                              sem_z).wait()
        plsc.subcore_barrier()
        return 0

    lax.fori_loop(0, C, channel, 0)


@jax.jit
def _softsplat_sc(ten2d, flow2d, mask2d):
    mesh = plsc.VectorSubcoreMesh(core_axis_name="c", subcore_axis_name="s")
    fn = pl.kernel(
        _sc_body,
        mesh=mesh,
        out_type=jax.ShapeDtypeStruct((B * C, HW), jnp.float32),
        scratch_types=[
            pltpu.VMEM_SHARED((PLANE,), jnp.float32),
            pltpu.VMEM_SHARED((PLANE,), jnp.float32),
            pltpu.VMEM((SLICE,), jnp.int32),
            pltpu.VMEM((SLICE,), jnp.float32),
            pltpu.VMEM((SLICE,), jnp.float32),
            pltpu.VMEM((SLICE,), jnp.float32),
            pltpu.VMEM((SLICE,), jnp.float32),
            pltpu.VMEM((2 * CH,), jnp.float32),
            *[pltpu.VMEM((CH,), jnp.int32) for _ in range(8)],
            *[pltpu.VMEM((CH,), jnp.float32) for _ in range(8)],
            pltpu.VMEM((CH,), jnp.float32),
            pltpu.SemaphoreType.DMA,
            pltpu.SemaphoreType.DMA,
            pltpu.SemaphoreType.DMA,
            pltpu.SemaphoreType.DMA,
            pltpu.SemaphoreType.DMA,
            pltpu.SemaphoreType.DMA,
            pltpu.SemaphoreType.DMA,
            pltpu.SemaphoreType.DMA,
            pltpu.SemaphoreType.DMA,
        ],
    )
    return fn(ten2d, flow2d, mask2d)


def kernel(tenInput, tenFlow, importance_mask):
    ten2d = tenInput.reshape(B * C, HW)
    flow2d = tenFlow.reshape(B * 2, HW)
    mask2d = importance_mask.reshape(B, HW)
    out = _softsplat_sc(ten2d, flow2d, mask2d)
    return out.reshape(B, C, H, W)
